# baseline (speedup 1.0000x reference)
import functools

import jax
import jax.numpy as jnp
from jax.experimental import pallas as pl
from jax.experimental.pallas import tpu as pltpu

B = 64
T = 200
AD = 128
ED = 128
OD = 2048

BB = 4

_DN = (((1,), (0,)), ((), ()))
_DNT = (((1,), (1,)), ((), ()))


def _fused_kernel(a_feat_ref, v_feat_ref, mask_a_ref, mask_v_ref,
                  modal_ref, time_ref, pad_ref,
                  Wa_ref, ba_ref, Wv_ref, bv_ref,
                  out_ref,
                  base_a_ref, base_v_ref, wpad_a_ref, wpad_v_ref,
                  Wa_bf_ref, Wv_bf_ref):
    b = pl.program_id(0)

    @pl.when(b == 0)
    def _():
        te = time_ref[...]
        ea = te + modal_ref[0:1, :]
        ev = te + modal_ref[1:2, :]
        Wae = Wa_ref[:, AD:]
        Wve = Wv_ref[:, AD:]
        base_a_ref[...] = (
            jax.lax.dot_general(ea, Wae, _DNT, preferred_element_type=jnp.float32)
            + ba_ref[...])
        base_v_ref[...] = (
            jax.lax.dot_general(ev, Wve, _DNT, preferred_element_type=jnp.float32)
            + bv_ref[...])
        wpad_a_ref[...] = jax.lax.dot_general(
            pad_ref[...], Wae, _DNT, preferred_element_type=jnp.float32)
        wpad_v_ref[...] = jax.lax.dot_general(
            pad_ref[...], Wve, _DNT, preferred_element_type=jnp.float32)
        Wa_bf_ref[...] = jnp.transpose(Wa_ref[:, :AD], (1, 0)).astype(jnp.bfloat16)
        Wv_bf_ref[...] = jnp.transpose(Wv_ref[:, :AD], (1, 0)).astype(jnp.bfloat16)

    a_feat = a_feat_ref[...].reshape(BB * T, AD).astype(jnp.bfloat16)
    v_feat = v_feat_ref[...].reshape(BB * T, AD).astype(jnp.bfloat16)
    a_mm = jax.lax.dot_general(a_feat, Wa_bf_ref[...], _DN,
                               preferred_element_type=jnp.float32)
    v_mm = jax.lax.dot_general(v_feat, Wv_bf_ref[...], _DN,
                               preferred_element_type=jnp.float32)
    for i in range(BB):
        mask_col_a = jnp.transpose(
            mask_a_ref[i].astype(jnp.float32), (1, 0))
        mask_col_v = jnp.transpose(
            mask_v_ref[i].astype(jnp.float32), (1, 0))
        out_ref[i, :T, :] = (a_mm[i * T:(i + 1) * T]
                             + base_a_ref[...]
                             + mask_col_a * wpad_a_ref[...])
        out_ref[i, T:, :] = (v_mm[i * T:(i + 1) * T]
                             + base_v_ref[...]
                             + mask_col_v * wpad_v_ref[...])


@jax.jit
def kernel(audio_feat, video_feat, mask_audio, mask_video, modal_emb,
           time_emb, pad_emb, W_audio, b_audio, W_video, b_video):
    mask_a = mask_audio.reshape(B, 1, T)
    mask_v = mask_video.reshape(B, 1, T)
    ba = b_audio.reshape(1, OD)
    bv = b_video.reshape(1, OD)

    out_tr = pl.pallas_call(
        _fused_kernel,
        grid=(B // BB,),
        in_specs=[
            pl.BlockSpec((BB, T, AD), lambda b: (b, 0, 0)),
            pl.BlockSpec((BB, T, AD), lambda b: (b, 0, 0)),
            pl.BlockSpec((BB, 1, T), lambda b: (b, 0, 0)),
            pl.BlockSpec((BB, 1, T), lambda b: (b, 0, 0)),
            pl.BlockSpec((2, ED), lambda b: (0, 0)),
            pl.BlockSpec((T, ED), lambda b: (0, 0)),
            pl.BlockSpec((1, ED), lambda b: (0, 0)),
            pl.BlockSpec((OD, AD + ED), lambda b: (0, 0)),
            pl.BlockSpec((1, OD), lambda b: (0, 0)),
            pl.BlockSpec((OD, AD + ED), lambda b: (0, 0)),
            pl.BlockSpec((1, OD), lambda b: (0, 0)),
        ],
        out_specs=pl.BlockSpec((BB, 2 * T, OD), lambda b: (b, 0, 0)),
        out_shape=jax.ShapeDtypeStruct((B, 2 * T, OD), jnp.float32),
        scratch_shapes=[
            pltpu.VMEM((T, OD), jnp.float32),
            pltpu.VMEM((T, OD), jnp.float32),
            pltpu.VMEM((1, OD), jnp.float32),
            pltpu.VMEM((1, OD), jnp.float32),
            pltpu.VMEM((AD, OD), jnp.bfloat16),
            pltpu.VMEM((AD, OD), jnp.bfloat16),
        ],
    )(audio_feat, video_feat, mask_a, mask_v, modal_emb, time_emb, pad_emb,
      W_audio, ba, W_video, bv)
    return jnp.transpose(out_tr, (0, 2, 1))


# speedup vs baseline: 1.0054x; 1.0054x over previous
import functools

import jax
import jax.numpy as jnp
from jax.experimental import pallas as pl
from jax.experimental.pallas import tpu as pltpu

B = 64
T = 200
AD = 128
ED = 128
OD = 2048

BB = 4

_DN = (((1,), (0,)), ((), ()))
_DNT = (((1,), (1,)), ((), ()))


def _fused_kernel(a_feat_ref, v_feat_ref, mask_a_ref, mask_v_ref,
                  modal_ref, time_ref, pad_ref,
                  Wa_ref, ba_ref, Wv_ref, bv_ref,
                  out_ref,
                  base_a_ref, base_v_ref, wpad_a_ref, wpad_v_ref,
                  Wa_bf_ref, Wv_bf_ref):
    b = pl.program_id(0)

    @pl.when(b == 0)
    def _():
        te = time_ref[...]
        ea = te + modal_ref[0:1, :]
        ev = te + modal_ref[1:2, :]
        Wae = Wa_ref[:, AD:]
        Wve = Wv_ref[:, AD:]
        base_a_ref[...] = (
            jax.lax.dot_general(ea, Wae, _DNT, preferred_element_type=jnp.float32)
            + ba_ref[...])
        base_v_ref[...] = (
            jax.lax.dot_general(ev, Wve, _DNT, preferred_element_type=jnp.float32)
            + bv_ref[...])
        wpad_a_ref[...] = jax.lax.dot_general(
            pad_ref[...], Wae, _DNT, preferred_element_type=jnp.float32)
        wpad_v_ref[...] = jax.lax.dot_general(
            pad_ref[...], Wve, _DNT, preferred_element_type=jnp.float32)
        Wa_bf_ref[...] = jnp.transpose(Wa_ref[:, :AD], (1, 0)).astype(jnp.bfloat16)
        Wv_bf_ref[...] = jnp.transpose(Wv_ref[:, :AD], (1, 0)).astype(jnp.bfloat16)

    a_feat = a_feat_ref[...].reshape(BB * T, AD).astype(jnp.bfloat16)
    v_feat = v_feat_ref[...].reshape(BB * T, AD).astype(jnp.bfloat16)
    a_mm = jax.lax.dot_general(a_feat, Wa_bf_ref[...], _DN,
                               preferred_element_type=jnp.float32)
    v_mm = jax.lax.dot_general(v_feat, Wv_bf_ref[...], _DN,
                               preferred_element_type=jnp.float32)
    for i in range(BB):
        row = b * BB + i
        mask_col_a = jnp.transpose(
            mask_a_ref[pl.ds(row, 1), :].astype(jnp.float32), (1, 0))
        mask_col_v = jnp.transpose(
            mask_v_ref[pl.ds(row, 1), :].astype(jnp.float32), (1, 0))
        out_ref[i, :T, :] = (a_mm[i * T:(i + 1) * T]
                             + base_a_ref[...]
                             + mask_col_a * wpad_a_ref[...])
        out_ref[i, T:, :] = (v_mm[i * T:(i + 1) * T]
                             + base_v_ref[...]
                             + mask_col_v * wpad_v_ref[...])


@jax.jit
def kernel(audio_feat, video_feat, mask_audio, mask_video, modal_emb,
           time_emb, pad_emb, W_audio, b_audio, W_video, b_video):
    ba = b_audio.reshape(1, OD)
    bv = b_video.reshape(1, OD)

    out_tr = pl.pallas_call(
        _fused_kernel,
        grid=(B // BB,),
        in_specs=[
            pl.BlockSpec((BB, T, AD), lambda b: (b, 0, 0)),
            pl.BlockSpec((BB, T, AD), lambda b: (b, 0, 0)),
            pl.BlockSpec((B, T), lambda b: (0, 0)),
            pl.BlockSpec((B, T), lambda b: (0, 0)),
            pl.BlockSpec((2, ED), lambda b: (0, 0)),
            pl.BlockSpec((T, ED), lambda b: (0, 0)),
            pl.BlockSpec((1, ED), lambda b: (0, 0)),
            pl.BlockSpec((OD, AD + ED), lambda b: (0, 0)),
            pl.BlockSpec((1, OD), lambda b: (0, 0)),
            pl.BlockSpec((OD, AD + ED), lambda b: (0, 0)),
            pl.BlockSpec((1, OD), lambda b: (0, 0)),
        ],
        out_specs=pl.BlockSpec((BB, 2 * T, OD), lambda b: (b, 0, 0)),
        out_shape=jax.ShapeDtypeStruct((B, 2 * T, OD), jnp.float32),
        scratch_shapes=[
            pltpu.VMEM((T, OD), jnp.float32),
            pltpu.VMEM((T, OD), jnp.float32),
            pltpu.VMEM((1, OD), jnp.float32),
            pltpu.VMEM((1, OD), jnp.float32),
            pltpu.VMEM((AD, OD), jnp.bfloat16),
            pltpu.VMEM((AD, OD), jnp.bfloat16),
        ],
    )(audio_feat, video_feat, mask_audio, mask_video, modal_emb, time_emb,
      pad_emb, W_audio, ba, W_video, bv)
    return jnp.transpose(out_tr, (0, 2, 1))
